# baseline (speedup 1.0000x reference)
import functools
import math

import jax
import jax.numpy as jnp
from jax import lax
from jax.experimental import pallas as pl
from jax.experimental.pallas import tpu as pltpu
from jax.experimental.pallas import tpu_sc as plsc

_N = 10000
_E = 320000
_D = 128
_EPS = 1e-5

_NC = 2
_NS = 16
_C = 80
_CHT = 252
_NCHUNKS = _NS * _CHT
_EPAD = _NCHUNKS * _C
_NR = 3
_NI = 4
_UNROLL = 12
_NACC = 10240
_ZROWS = _NACC // _NS
_DW = 16


def _sc_segment_sum(feat, edge_r, with_deg):
    mesh = plsc.VectorSubcoreMesh(core_axis_name="c", subcore_axis_name="s",
                                  num_cores=_NC, num_subcores=_NS)

    def body(feat_hbm, edge_hbm, *refs):
        if with_deg:
            (acc_out, deg_out, i0, i1, i2, i3, d0, d1, d2, d3,
             r0, r1, r2, ones_v, acc_sh, deg_sh,
             si0, si1, si2, si3, sr0, sr1, sr2) = refs
        else:
            (acc_out, i0, i1, i2, i3, d0, d1, d2, d3,
             r0, r1, r2, ones_v, acc_sh, deg_sh,
             si0, si1, si2, si3, sr0, sr1, sr2) = refs
            deg_out = None
        cid = lax.axis_index("c")
        sid = lax.axis_index("s")
        idx = [i0, i1, i2, i3]
        dsti = [d0, d1, d2, d3]
        isem = [si0, si1, si2, si3]
        rows = [r0, r1, r2]
        rsem = [sr0, sr1, sr2]
        cbase = sid * _CHT
        base = sid * _ZROWS

        @pl.when(cid == 0)
        def _sc0():
            def zrow(r, carry):
                for c in range(_D // 16):
                    r0[r, pl.ds(c * 16, 16)] = jnp.zeros((16,),
                                                         jnp.float32)
                return carry
            lax.fori_loop(0, _C, zrow, 0)
            for j in range(_ZROWS // _C):
                pltpu.sync_copy(r0, acc_sh.at[pl.ds(base + j * _C, _C)])
            _rem = _ZROWS % _C
            if _rem:
                pltpu.sync_copy(
                    r0.at[pl.ds(0, _rem)],
                    acc_sh.at[pl.ds(base + (_ZROWS // _C) * _C, _rem)])
            plsc.subcore_barrier()

            def load_idx(j, slot):
                pltpu.async_copy(
                    edge_hbm.at[:, pl.ds((cbase + j) * _C, _C)],
                    idx[slot], isem[slot])

            def wait_idx(j, slot):
                pltpu.make_async_copy(
                    edge_hbm.at[:, pl.ds((cbase + j) * _C, _C)],
                    idx[slot], isem[slot]).wait()

            def gather(slot_i, slot_r):
                pltpu.async_copy(feat_hbm.at[idx[slot_i].at[0]],
                                 rows[slot_r], rsem[slot_r])

            def wait_gather(slot_i, slot_r):
                pltpu.make_async_copy(feat_hbm.at[idx[slot_i].at[0]],
                                      rows[slot_r], rsem[slot_r]).wait()

            for j in range(_NI - 1):
                load_idx(j, j)
            for j in range(_NR - 1):
                wait_idx(j, j)
                gather(j, j)

            def step(it, carry):
                for k in range(_UNROLL):
                    jv = it * _UNROLL + k
                    s_i = k % _NI
                    s_g = (k + _NR - 1) % _NI
                    s_l = (k + _NI - 1) % _NI
                    s_r = k % _NR

                    wait_gather(s_i, s_r)

                    @pl.when(jv + _NR - 1 < _CHT)
                    def _():
                        wait_idx(jv + _NR - 1, s_g)
                        gather(s_g, (k + _NR - 1) % _NR)

                    @pl.when(jv + _NI - 1 < _CHT)
                    def _():
                        load_idx(jv + _NI - 1, s_l)

                    pltpu.sync_copy(rows[s_r], acc_sh.at[idx[s_i].at[1]],
                                    add=True)
                return carry
            lax.fori_loop(0, _CHT // _UNROLL, step, 0)
            plsc.subcore_barrier()

            pltpu.sync_copy(acc_sh.at[pl.ds(base, _ZROWS)],
                            acc_out.at[pl.ds(base, _ZROWS)])

        if with_deg:
            @pl.when(cid == 1)
            def _sc1():
                def fillrow(val):
                    def f(r, carry):
                        ones_v[r, pl.ds(0, _DW)] = jnp.full((_DW,), val,
                                                            jnp.float32)
                        return carry
                    return f
                lax.fori_loop(0, _C, fillrow(0.0), 0)
                for j in range(_ZROWS // _C):
                    pltpu.sync_copy(ones_v,
                                    deg_sh.at[pl.ds(base + j * _C, _C)])
                _rem = _ZROWS % _C
                if _rem:
                    pltpu.sync_copy(
                        ones_v.at[pl.ds(0, _rem)],
                        deg_sh.at[pl.ds(base + (_ZROWS // _C) * _C, _rem)])
                lax.fori_loop(0, _C, fillrow(1.0), 0)
                plsc.subcore_barrier()

                def load_d(j, slot):
                    pltpu.async_copy(
                        edge_hbm.at[pl.ds(1, 1), pl.ds((cbase + j) * _C, _C)],
                        dsti[slot], isem[slot])

                def wait_d(j, slot):
                    pltpu.make_async_copy(
                        edge_hbm.at[pl.ds(1, 1), pl.ds((cbase + j) * _C, _C)],
                        dsti[slot], isem[slot]).wait()

                for j in range(_NI - 1):
                    load_d(j, j)

                def dstep(it, carry):
                    for k in range(_NI):
                        jv = it * _NI + k
                        s_i = k % _NI
                        s_l = (k + _NI - 1) % _NI
                        wait_d(jv, s_i)

                        @pl.when(jv + _NI - 1 < _CHT)
                        def _():
                            load_d(jv + _NI - 1, s_l)

                        pltpu.sync_copy(ones_v,
                                        deg_sh.at[dsti[s_i].at[0]],
                                        add=True)
                    return carry
                lax.fori_loop(0, _CHT // _NI, dstep, 0)
                plsc.subcore_barrier()

                pltpu.sync_copy(deg_sh.at[pl.ds(base, _ZROWS)],
                                deg_out.at[pl.ds(base, _ZROWS)])

    if with_deg:
        out_type = [jax.ShapeDtypeStruct((_NACC, _D), jnp.float32),
                    jax.ShapeDtypeStruct((_NACC, _DW), jnp.float32)]
    else:
        out_type = [jax.ShapeDtypeStruct((_NACC, _D), jnp.float32)]

    return pl.kernel(
        body,
        out_type=out_type,
        mesh=mesh,
        scratch_types=[
            pltpu.VMEM((2, _C), jnp.int32),
            pltpu.VMEM((2, _C), jnp.int32),
            pltpu.VMEM((2, _C), jnp.int32),
            pltpu.VMEM((2, _C), jnp.int32),
            pltpu.VMEM((1, _C), jnp.int32),
            pltpu.VMEM((1, _C), jnp.int32),
            pltpu.VMEM((1, _C), jnp.int32),
            pltpu.VMEM((1, _C), jnp.int32),
            pltpu.VMEM((_C, _D), jnp.float32),
            pltpu.VMEM((_C, _D), jnp.float32),
            pltpu.VMEM((_C, _D), jnp.float32),
            pltpu.VMEM((_C, _DW), jnp.float32),
            pltpu.VMEM_SHARED((_NACC, _D), jnp.float32),
            pltpu.VMEM_SHARED((_NACC, _DW), jnp.float32),
            pltpu.SemaphoreType.DMA,
            pltpu.SemaphoreType.DMA,
            pltpu.SemaphoreType.DMA,
            pltpu.SemaphoreType.DMA,
            pltpu.SemaphoreType.DMA,
            pltpu.SemaphoreType.DMA,
            pltpu.SemaphoreType.DMA,
        ],
        compiler_params=pltpu.CompilerParams(use_tc_tiling_on_sc=False),
    )(feat, edge_r)


_BLK = 400
_INV_STD = 1.0 / math.sqrt(1.0 + _EPS)


def _l1_body(p_ref, deg_ref, x_ref, wn_ref, wr_ref, b_ref, g_ref, be_ref,
             h_ref):
    rdeg = 1.0 / jnp.maximum(deg_ref[:, 0:1], 1.0)
    mean = p_ref[...] * rdeg
    z = (jnp.dot(mean, wn_ref[...], preferred_element_type=jnp.float32)
         + jnp.dot(x_ref[...], wr_ref[...], preferred_element_type=jnp.float32)
         + b_ref[...])
    z = z * (_INV_STD * g_ref[...]) + be_ref[...]
    h_ref[...] = jnp.maximum(z, 0.0)


def _tc_layer1(p, deg, x, w1nT, w1rT, b1, gamma1, beta1):
    return pl.pallas_call(
        _l1_body,
        grid=(_N // _BLK,),
        in_specs=[
            pl.BlockSpec((_BLK, _D), lambda i: (i, 0)),
            pl.BlockSpec((_BLK, _DW), lambda i: (i, 0)),
            pl.BlockSpec((_BLK, _D), lambda i: (i, 0)),
            pl.BlockSpec((_D, _D), lambda i: (0, 0)),
            pl.BlockSpec((_D, _D), lambda i: (0, 0)),
            pl.BlockSpec((1, _D), lambda i: (0, 0)),
            pl.BlockSpec((1, _D), lambda i: (0, 0)),
            pl.BlockSpec((1, _D), lambda i: (0, 0)),
        ],
        out_specs=pl.BlockSpec((_BLK, _D), lambda i: (i, 0)),
        out_shape=jax.ShapeDtypeStruct((_N, _D), jnp.float32),
    )(p, deg, x, w1nT, w1rT, b1, gamma1, beta1)


def _l2_body(p_ref, h_ref, deg_ref, wn_ref, wr_ref, b_ref, out_ref):
    rdeg = 1.0 / jnp.maximum(deg_ref[:, 0:1], 1.0)
    mean = p_ref[...] * rdeg
    z = (jnp.dot(mean, wn_ref[...], preferred_element_type=jnp.float32)
         + jnp.dot(h_ref[...], wr_ref[...], preferred_element_type=jnp.float32)
         + b_ref[...])
    m = jnp.max(z, axis=1, keepdims=True)
    s = jnp.sum(jnp.exp(z - m), axis=1, keepdims=True)
    out_ref[...] = z - m - jnp.log(s)


def _tc_layer2(p2, h, deg, w2nT, w2rT, b2):
    return pl.pallas_call(
        _l2_body,
        grid=(_N // _BLK,),
        in_specs=[
            pl.BlockSpec((_BLK, _D), lambda i: (i, 0)),
            pl.BlockSpec((_BLK, _D), lambda i: (i, 0)),
            pl.BlockSpec((_BLK, _DW), lambda i: (i, 0)),
            pl.BlockSpec((_D, _D), lambda i: (0, 0)),
            pl.BlockSpec((_D, _D), lambda i: (0, 0)),
            pl.BlockSpec((1, _D), lambda i: (0, 0)),
        ],
        out_specs=pl.BlockSpec((_BLK, _D), lambda i: (i, 0)),
        out_shape=jax.ShapeDtypeStruct((_N, _D), jnp.float32),
    )(p2, h, deg, w2nT, w2rT, b2)


def kernel(x, edge_index, W1n, W1r, b1, gamma1, beta1, W2n, W2r, b2):
    pad = _EPAD - _E
    pad_cols = jnp.stack([jnp.zeros((pad,), jnp.int32),
                          jnp.full((pad,), _N, jnp.int32)])
    edge_r = jnp.concatenate([edge_index, pad_cols], axis=1)

    p1, deg = _sc_segment_sum(x, edge_r, True)
    h = _tc_layer1(p1, deg, x, W1n.T, W1r.T, b1.reshape(1, _D),
                   gamma1.reshape(1, _D), beta1.reshape(1, _D))
    (p2,) = _sc_segment_sum(h, edge_r, False)
    return _tc_layer2(p2, h, deg, W2n.T, W2r.T, b2.reshape(1, _D))


# speedup vs baseline: 1.3201x; 1.3201x over previous
import functools
import math

import jax
import jax.numpy as jnp
from jax import lax
from jax.experimental import pallas as pl
from jax.experimental.pallas import tpu as pltpu
from jax.experimental.pallas import tpu_sc as plsc

_N = 10000
_E = 320000
_D = 128
_EPS = 1e-5

_NC = 2
_NS = 16
_C = 88
_CHT = 228
_NCHUNKS = _NS * _CHT
_EPAD = _NCHUNKS * _C
_NR = 3
_NI = 4
_UNROLL = 12
_NACC = 10240
_ZROWS = _NACC // _NS
_DW = 16


def _sc_segment_sum(feat, edge_r, with_deg):
    mesh = plsc.VectorSubcoreMesh(core_axis_name="c", subcore_axis_name="s",
                                  num_cores=_NC, num_subcores=_NS)

    def body(feat_hbm, edge_hbm, *refs):
        if with_deg:
            (acc_out, deg_out, i0, i1, i2, i3, d0, d1, d2, d3,
             r0, r1, r2, ones_v, acc_sh, deg_sh,
             si0, si1, si2, si3, sr0, sr1, sr2) = refs
        else:
            (acc_out, i0, i1, i2, i3, d0, d1, d2, d3,
             r0, r1, r2, ones_v, acc_sh, deg_sh,
             si0, si1, si2, si3, sr0, sr1, sr2) = refs
            deg_out = None
        cid = lax.axis_index("c")
        sid = lax.axis_index("s")
        idx = [i0, i1, i2, i3]
        dsti = [d0, d1, d2, d3]
        isem = [si0, si1, si2, si3]
        rows = [r0, r1, r2]
        rsem = [sr0, sr1, sr2]
        cbase = sid * _CHT
        base = sid * _ZROWS

        @pl.when(cid == 0)
        def _sc0():
            def zrow(r, carry):
                for c in range(_D // 16):
                    r0[r, pl.ds(c * 16, 16)] = jnp.zeros((16,),
                                                         jnp.float32)
                return carry
            lax.fori_loop(0, _C, zrow, 0)
            for j in range(_ZROWS // _C):
                pltpu.sync_copy(r0, acc_sh.at[pl.ds(base + j * _C, _C)])
            _rem = _ZROWS % _C
            if _rem:
                pltpu.sync_copy(
                    r0.at[pl.ds(0, _rem)],
                    acc_sh.at[pl.ds(base + (_ZROWS // _C) * _C, _rem)])
            plsc.subcore_barrier()

            def load_idx(j, slot):
                pltpu.async_copy(
                    edge_hbm.at[:, pl.ds((cbase + j) * _C, _C)],
                    idx[slot], isem[slot])

            def wait_idx(j, slot):
                pltpu.make_async_copy(
                    edge_hbm.at[:, pl.ds((cbase + j) * _C, _C)],
                    idx[slot], isem[slot]).wait()

            def gather(slot_i, slot_r):
                pltpu.async_copy(feat_hbm.at[idx[slot_i].at[0]],
                                 rows[slot_r], rsem[slot_r])

            def wait_gather(slot_i, slot_r):
                pltpu.make_async_copy(feat_hbm.at[idx[slot_i].at[0]],
                                      rows[slot_r], rsem[slot_r]).wait()

            for j in range(_NI - 1):
                load_idx(j, j)
            for j in range(_NR - 1):
                wait_idx(j, j)
                gather(j, j)

            def step(it, carry):
                for k in range(_UNROLL):
                    jv = it * _UNROLL + k
                    s_i = k % _NI
                    s_g = (k + _NR - 1) % _NI
                    s_l = (k + _NI - 1) % _NI
                    s_r = k % _NR

                    wait_gather(s_i, s_r)

                    @pl.when(jv + _NR - 1 < _CHT)
                    def _():
                        wait_idx(jv + _NR - 1, s_g)
                        gather(s_g, (k + _NR - 1) % _NR)

                    @pl.when(jv + _NI - 1 < _CHT)
                    def _():
                        load_idx(jv + _NI - 1, s_l)

                    pltpu.sync_copy(rows[s_r], acc_sh.at[idx[s_i].at[1]],
                                    add=True)
                return carry
            lax.fori_loop(0, _CHT // _UNROLL, step, 0)
            plsc.subcore_barrier()

            pltpu.sync_copy(acc_sh.at[pl.ds(base, _ZROWS)],
                            acc_out.at[pl.ds(base, _ZROWS)])

        if with_deg:
            @pl.when(cid == 1)
            def _sc1():
                def fillrow(val):
                    def f(r, carry):
                        ones_v[r, pl.ds(0, _DW)] = jnp.full((_DW,), val,
                                                            jnp.float32)
                        return carry
                    return f
                lax.fori_loop(0, _C, fillrow(0.0), 0)
                for j in range(_ZROWS // _C):
                    pltpu.sync_copy(ones_v,
                                    deg_sh.at[pl.ds(base + j * _C, _C)])
                _rem = _ZROWS % _C
                if _rem:
                    pltpu.sync_copy(
                        ones_v.at[pl.ds(0, _rem)],
                        deg_sh.at[pl.ds(base + (_ZROWS // _C) * _C, _rem)])
                lax.fori_loop(0, _C, fillrow(1.0), 0)
                plsc.subcore_barrier()

                def load_d(j, slot):
                    pltpu.async_copy(
                        edge_hbm.at[pl.ds(1, 1), pl.ds((cbase + j) * _C, _C)],
                        dsti[slot], isem[slot])

                def wait_d(j, slot):
                    pltpu.make_async_copy(
                        edge_hbm.at[pl.ds(1, 1), pl.ds((cbase + j) * _C, _C)],
                        dsti[slot], isem[slot]).wait()

                for j in range(_NI - 1):
                    load_d(j, j)

                def dstep(it, carry):
                    for k in range(_NI):
                        jv = it * _NI + k
                        s_i = k % _NI
                        s_l = (k + _NI - 1) % _NI
                        wait_d(jv, s_i)

                        @pl.when(jv + _NI - 1 < _CHT)
                        def _():
                            load_d(jv + _NI - 1, s_l)

                        pltpu.sync_copy(ones_v,
                                        deg_sh.at[dsti[s_i].at[0]],
                                        add=True)
                    return carry
                lax.fori_loop(0, _CHT // _NI, dstep, 0)
                plsc.subcore_barrier()

                pltpu.sync_copy(deg_sh.at[pl.ds(base, _ZROWS)],
                                deg_out.at[pl.ds(base, _ZROWS)])

    if with_deg:
        out_type = [jax.ShapeDtypeStruct((_NACC, _D), jnp.float32),
                    jax.ShapeDtypeStruct((_NACC, _DW), jnp.float32)]
    else:
        out_type = [jax.ShapeDtypeStruct((_NACC, _D), jnp.float32)]

    return pl.kernel(
        body,
        out_type=out_type,
        mesh=mesh,
        scratch_types=[
            pltpu.VMEM((2, _C), jnp.int32),
            pltpu.VMEM((2, _C), jnp.int32),
            pltpu.VMEM((2, _C), jnp.int32),
            pltpu.VMEM((2, _C), jnp.int32),
            pltpu.VMEM((1, _C), jnp.int32),
            pltpu.VMEM((1, _C), jnp.int32),
            pltpu.VMEM((1, _C), jnp.int32),
            pltpu.VMEM((1, _C), jnp.int32),
            pltpu.VMEM((_C, _D), jnp.float32),
            pltpu.VMEM((_C, _D), jnp.float32),
            pltpu.VMEM((_C, _D), jnp.float32),
            pltpu.VMEM((_C, _DW), jnp.float32),
            pltpu.VMEM_SHARED((_NACC, _D), jnp.float32),
            pltpu.VMEM_SHARED((_NACC, _DW), jnp.float32),
            pltpu.SemaphoreType.DMA,
            pltpu.SemaphoreType.DMA,
            pltpu.SemaphoreType.DMA,
            pltpu.SemaphoreType.DMA,
            pltpu.SemaphoreType.DMA,
            pltpu.SemaphoreType.DMA,
            pltpu.SemaphoreType.DMA,
        ],
        compiler_params=pltpu.CompilerParams(use_tc_tiling_on_sc=False),
    )(feat, edge_r)


_BLK = 1000
_INV_STD = 1.0 / math.sqrt(1.0 + _EPS)


def _l1_body(p_ref, deg_ref, x_ref, wn_ref, wr_ref, b_ref, g_ref, be_ref,
             h_ref):
    rdeg = 1.0 / jnp.maximum(deg_ref[:, 0:1], 1.0)
    mean = p_ref[...] * rdeg
    z = (jnp.dot(mean, wn_ref[...], preferred_element_type=jnp.float32)
         + jnp.dot(x_ref[...], wr_ref[...], preferred_element_type=jnp.float32)
         + b_ref[...])
    z = z * (_INV_STD * g_ref[...]) + be_ref[...]
    h_ref[...] = jnp.maximum(z, 0.0)


def _tc_layer1(p, deg, x, w1nT, w1rT, b1, gamma1, beta1):
    return pl.pallas_call(
        _l1_body,
        grid=(_N // _BLK,),
        in_specs=[
            pl.BlockSpec((_BLK, _D), lambda i: (i, 0)),
            pl.BlockSpec((_BLK, _DW), lambda i: (i, 0)),
            pl.BlockSpec((_BLK, _D), lambda i: (i, 0)),
            pl.BlockSpec((_D, _D), lambda i: (0, 0)),
            pl.BlockSpec((_D, _D), lambda i: (0, 0)),
            pl.BlockSpec((1, _D), lambda i: (0, 0)),
            pl.BlockSpec((1, _D), lambda i: (0, 0)),
            pl.BlockSpec((1, _D), lambda i: (0, 0)),
        ],
        out_specs=pl.BlockSpec((_BLK, _D), lambda i: (i, 0)),
        out_shape=jax.ShapeDtypeStruct((_N, _D), jnp.float32),
    )(p, deg, x, w1nT, w1rT, b1, gamma1, beta1)


def _l2_body(p_ref, h_ref, deg_ref, wn_ref, wr_ref, b_ref, out_ref):
    rdeg = 1.0 / jnp.maximum(deg_ref[:, 0:1], 1.0)
    mean = p_ref[...] * rdeg
    z = (jnp.dot(mean, wn_ref[...], preferred_element_type=jnp.float32)
         + jnp.dot(h_ref[...], wr_ref[...], preferred_element_type=jnp.float32)
         + b_ref[...])
    m = jnp.max(z, axis=1, keepdims=True)
    s = jnp.sum(jnp.exp(z - m), axis=1, keepdims=True)
    out_ref[...] = z - m - jnp.log(s)


def _tc_layer2(p2, h, deg, w2nT, w2rT, b2):
    return pl.pallas_call(
        _l2_body,
        grid=(_N // _BLK,),
        in_specs=[
            pl.BlockSpec((_BLK, _D), lambda i: (i, 0)),
            pl.BlockSpec((_BLK, _D), lambda i: (i, 0)),
            pl.BlockSpec((_BLK, _DW), lambda i: (i, 0)),
            pl.BlockSpec((_D, _D), lambda i: (0, 0)),
            pl.BlockSpec((_D, _D), lambda i: (0, 0)),
            pl.BlockSpec((1, _D), lambda i: (0, 0)),
        ],
        out_specs=pl.BlockSpec((_BLK, _D), lambda i: (i, 0)),
        out_shape=jax.ShapeDtypeStruct((_N, _D), jnp.float32),
    )(p2, h, deg, w2nT, w2rT, b2)


def kernel(x, edge_index, W1n, W1r, b1, gamma1, beta1, W2n, W2r, b2):
    pad = _EPAD - _E
    pad_cols = jnp.stack([jnp.zeros((pad,), jnp.int32),
                          jnp.full((pad,), _N, jnp.int32)])
    edge_r = jnp.concatenate([edge_index, pad_cols], axis=1)

    p1, deg = _sc_segment_sum(x, edge_r, True)
    h = _tc_layer1(p1, deg, x, W1n.T, W1r.T, b1.reshape(1, _D),
                   gamma1.reshape(1, _D), beta1.reshape(1, _D))
    (p2,) = _sc_segment_sum(h, edge_r, False)
    return _tc_layer2(p2, h, deg, W2n.T, W2r.T, b2.reshape(1, _D))


# speedup vs baseline: 1.3353x; 1.0115x over previous
import functools
import math

import jax
import jax.numpy as jnp
from jax import lax
from jax.experimental import pallas as pl
from jax.experimental.pallas import tpu as pltpu
from jax.experimental.pallas import tpu_sc as plsc

_N = 10000
_E = 320000
_D = 128
_EPS = 1e-5

_NC = 2
_NS = 16
_C = 88
_CHT = 228
_NCHUNKS = _NS * _CHT
_EPAD = _NCHUNKS * _C
_NR = 3
_NI = 4
_UNROLL = 12
_NACC = 10240
_ZROWS = _NACC // _NS
_DW = 16


def _sc_segment_sum(feat, edge_r, with_deg):
    mesh = plsc.VectorSubcoreMesh(core_axis_name="c", subcore_axis_name="s",
                                  num_cores=_NC, num_subcores=_NS)

    def body(feat_hbm, edge_hbm, *refs):
        if with_deg:
            (acc_out, deg_out, i0, i1, i2, i3, d0, d1, d2, d3,
             r0, r1, r2, ones_v, acc_sh, deg_sh,
             si0, si1, si2, si3, sr0, sr1, sr2) = refs
        else:
            (acc_out, i0, i1, i2, i3, d0, d1, d2, d3,
             r0, r1, r2, ones_v, acc_sh, deg_sh,
             si0, si1, si2, si3, sr0, sr1, sr2) = refs
            deg_out = None
        cid = lax.axis_index("c")
        sid = lax.axis_index("s")
        idx = [i0, i1, i2, i3]
        dsti = [d0, d1, d2, d3]
        isem = [si0, si1, si2, si3]
        rows = [r0, r1, r2]
        rsem = [sr0, sr1, sr2]
        cbase = sid * _CHT
        base = sid * _ZROWS

        @pl.when(cid == 0)
        def _sc0():
            def zrow(r, carry):
                for c in range(_D // 16):
                    r0[r, pl.ds(c * 16, 16)] = jnp.zeros((16,),
                                                         jnp.float32)
                return carry
            lax.fori_loop(0, _C, zrow, 0)
            for j in range(_ZROWS // _C):
                pltpu.sync_copy(r0, acc_sh.at[pl.ds(base + j * _C, _C)])
            _rem = _ZROWS % _C
            if _rem:
                pltpu.sync_copy(
                    r0.at[pl.ds(0, _rem)],
                    acc_sh.at[pl.ds(base + (_ZROWS // _C) * _C, _rem)])
            plsc.subcore_barrier()

            def load_idx(j, slot):
                pltpu.async_copy(
                    edge_hbm.at[:, pl.ds((cbase + j) * _C, _C)],
                    idx[slot], isem[slot])

            def wait_idx(j, slot):
                pltpu.make_async_copy(
                    edge_hbm.at[:, pl.ds((cbase + j) * _C, _C)],
                    idx[slot], isem[slot]).wait()

            def gather(slot_i, slot_r):
                pltpu.async_copy(feat_hbm.at[idx[slot_i].at[0]],
                                 rows[slot_r], rsem[slot_r])

            def wait_gather(slot_i, slot_r):
                pltpu.make_async_copy(feat_hbm.at[idx[slot_i].at[0]],
                                      rows[slot_r], rsem[slot_r]).wait()

            for j in range(_NI - 1):
                load_idx(j, j)
            for j in range(_NR - 1):
                wait_idx(j, j)
                gather(j, j)

            def step(it, carry):
                for k in range(_UNROLL):
                    jv = it * _UNROLL + k
                    s_i = k % _NI
                    s_g = (k + _NR - 1) % _NI
                    s_l = (k + _NI - 1) % _NI
                    s_r = k % _NR

                    wait_gather(s_i, s_r)

                    @pl.when(jv + _NR - 1 < _CHT)
                    def _():
                        wait_idx(jv + _NR - 1, s_g)
                        gather(s_g, (k + _NR - 1) % _NR)

                    @pl.when(jv + _NI - 1 < _CHT)
                    def _():
                        load_idx(jv + _NI - 1, s_l)

                    pltpu.sync_copy(rows[s_r], acc_sh.at[idx[s_i].at[1]],
                                    add=True)
                return carry
            lax.fori_loop(0, _CHT // _UNROLL, step, 0)
            plsc.subcore_barrier()

            pltpu.sync_copy(acc_sh.at[pl.ds(base, _ZROWS)],
                            acc_out.at[pl.ds(base, _ZROWS)])

        if with_deg:
            @pl.when(cid == 1)
            def _sc1():
                def fillrow(val):
                    def f(r, carry):
                        ones_v[r, pl.ds(0, _DW)] = jnp.full((_DW,), val,
                                                            jnp.float32)
                        return carry
                    return f
                lax.fori_loop(0, _C, fillrow(0.0), 0)
                for j in range(_ZROWS // _C):
                    pltpu.sync_copy(ones_v,
                                    deg_sh.at[pl.ds(base + j * _C, _C)])
                _rem = _ZROWS % _C
                if _rem:
                    pltpu.sync_copy(
                        ones_v.at[pl.ds(0, _rem)],
                        deg_sh.at[pl.ds(base + (_ZROWS // _C) * _C, _rem)])
                lax.fori_loop(0, _C, fillrow(1.0), 0)
                plsc.subcore_barrier()

                def load_d(j, slot):
                    pltpu.async_copy(
                        edge_hbm.at[pl.ds(1, 1), pl.ds((cbase + j) * _C, _C)],
                        dsti[slot], isem[slot])

                def wait_d(j, slot):
                    pltpu.make_async_copy(
                        edge_hbm.at[pl.ds(1, 1), pl.ds((cbase + j) * _C, _C)],
                        dsti[slot], isem[slot]).wait()

                for j in range(_NI - 1):
                    load_d(j, j)

                def dstep(it, carry):
                    for k in range(_NI):
                        jv = it * _NI + k
                        s_i = k % _NI
                        s_l = (k + _NI - 1) % _NI
                        wait_d(jv, s_i)

                        @pl.when(jv + _NI - 1 < _CHT)
                        def _():
                            load_d(jv + _NI - 1, s_l)

                        pltpu.sync_copy(ones_v,
                                        deg_sh.at[dsti[s_i].at[0]],
                                        add=True)
                    return carry
                lax.fori_loop(0, _CHT // _NI, dstep, 0)
                plsc.subcore_barrier()

                pltpu.sync_copy(deg_sh.at[pl.ds(base, _ZROWS)],
                                deg_out.at[pl.ds(base, _ZROWS)])

    if with_deg:
        out_type = [jax.ShapeDtypeStruct((_NACC, _D), jnp.float32),
                    jax.ShapeDtypeStruct((_NACC, _DW), jnp.float32)]
    else:
        out_type = [jax.ShapeDtypeStruct((_NACC, _D), jnp.float32)]

    return pl.kernel(
        body,
        out_type=out_type,
        mesh=mesh,
        scratch_types=[
            pltpu.VMEM((2, _C), jnp.int32),
            pltpu.VMEM((2, _C), jnp.int32),
            pltpu.VMEM((2, _C), jnp.int32),
            pltpu.VMEM((2, _C), jnp.int32),
            pltpu.VMEM((1, _C), jnp.int32),
            pltpu.VMEM((1, _C), jnp.int32),
            pltpu.VMEM((1, _C), jnp.int32),
            pltpu.VMEM((1, _C), jnp.int32),
            pltpu.VMEM((_C, _D), jnp.float32),
            pltpu.VMEM((_C, _D), jnp.float32),
            pltpu.VMEM((_C, _D), jnp.float32),
            pltpu.VMEM((_C, _DW), jnp.float32),
            pltpu.VMEM_SHARED((_NACC, _D), jnp.float32),
            pltpu.VMEM_SHARED((_NACC, _DW), jnp.float32),
            pltpu.SemaphoreType.DMA,
            pltpu.SemaphoreType.DMA,
            pltpu.SemaphoreType.DMA,
            pltpu.SemaphoreType.DMA,
            pltpu.SemaphoreType.DMA,
            pltpu.SemaphoreType.DMA,
            pltpu.SemaphoreType.DMA,
        ],
        compiler_params=pltpu.CompilerParams(use_tc_tiling_on_sc=False),
    )(feat, edge_r)


_BLK = 2000
_INV_STD = 1.0 / math.sqrt(1.0 + _EPS)


def _l1_body(p_ref, deg_ref, x_ref, wn_ref, wr_ref, b_ref, g_ref, be_ref,
             h_ref):
    rdeg = 1.0 / jnp.maximum(deg_ref[:, 0:1], 1.0)
    mean = p_ref[...] * rdeg
    z = (jnp.dot(mean, wn_ref[...], preferred_element_type=jnp.float32)
         + jnp.dot(x_ref[...], wr_ref[...], preferred_element_type=jnp.float32)
         + b_ref[...])
    z = z * (_INV_STD * g_ref[...]) + be_ref[...]
    h_ref[...] = jnp.maximum(z, 0.0)


def _tc_layer1(p, deg, x, w1nT, w1rT, b1, gamma1, beta1):
    return pl.pallas_call(
        _l1_body,
        grid=(_N // _BLK,),
        in_specs=[
            pl.BlockSpec((_BLK, _D), lambda i: (i, 0)),
            pl.BlockSpec((_BLK, _DW), lambda i: (i, 0)),
            pl.BlockSpec((_BLK, _D), lambda i: (i, 0)),
            pl.BlockSpec((_D, _D), lambda i: (0, 0)),
            pl.BlockSpec((_D, _D), lambda i: (0, 0)),
            pl.BlockSpec((1, _D), lambda i: (0, 0)),
            pl.BlockSpec((1, _D), lambda i: (0, 0)),
            pl.BlockSpec((1, _D), lambda i: (0, 0)),
        ],
        out_specs=pl.BlockSpec((_BLK, _D), lambda i: (i, 0)),
        out_shape=jax.ShapeDtypeStruct((_N, _D), jnp.float32),
    )(p, deg, x, w1nT, w1rT, b1, gamma1, beta1)


def _l2_body(p_ref, h_ref, deg_ref, wn_ref, wr_ref, b_ref, out_ref):
    rdeg = 1.0 / jnp.maximum(deg_ref[:, 0:1], 1.0)
    mean = p_ref[...] * rdeg
    z = (jnp.dot(mean, wn_ref[...], preferred_element_type=jnp.float32)
         + jnp.dot(h_ref[...], wr_ref[...], preferred_element_type=jnp.float32)
         + b_ref[...])
    m = jnp.max(z, axis=1, keepdims=True)
    s = jnp.sum(jnp.exp(z - m), axis=1, keepdims=True)
    out_ref[...] = z - m - jnp.log(s)


def _tc_layer2(p2, h, deg, w2nT, w2rT, b2):
    return pl.pallas_call(
        _l2_body,
        grid=(_N // _BLK,),
        in_specs=[
            pl.BlockSpec((_BLK, _D), lambda i: (i, 0)),
            pl.BlockSpec((_BLK, _D), lambda i: (i, 0)),
            pl.BlockSpec((_BLK, _DW), lambda i: (i, 0)),
            pl.BlockSpec((_D, _D), lambda i: (0, 0)),
            pl.BlockSpec((_D, _D), lambda i: (0, 0)),
            pl.BlockSpec((1, _D), lambda i: (0, 0)),
        ],
        out_specs=pl.BlockSpec((_BLK, _D), lambda i: (i, 0)),
        out_shape=jax.ShapeDtypeStruct((_N, _D), jnp.float32),
    )(p2, h, deg, w2nT, w2rT, b2)


def kernel(x, edge_index, W1n, W1r, b1, gamma1, beta1, W2n, W2r, b2):
    pad = _EPAD - _E
    pad_cols = jnp.stack([jnp.zeros((pad,), jnp.int32),
                          jnp.full((pad,), _N, jnp.int32)])
    edge_r = jnp.concatenate([edge_index, pad_cols], axis=1)

    p1, deg = _sc_segment_sum(x, edge_r, True)
    h = _tc_layer1(p1, deg, x, W1n.T, W1r.T, b1.reshape(1, _D),
                   gamma1.reshape(1, _D), beta1.reshape(1, _D))
    (p2,) = _sc_segment_sum(h, edge_r, False)
    return _tc_layer2(p2, h, deg, W2n.T, W2r.T, b2.reshape(1, _D))
